# Initial kernel scaffold; baseline (speedup 1.0000x reference)
import jax
import jax.numpy as jnp
from jax.experimental import pallas as pl


def kernel(input, target):
    raise NotImplementedError("write your pallas kernel here")


# speedup vs baseline: 179.5472x; 179.5472x over previous
import functools

import jax
import jax.numpy as jnp
from jax import lax
from jax.experimental import pallas as pl
from jax.experimental.pallas import tpu as pltpu

_NCLS = 19
_BY = 64


def _rce_kernel(x_ref, t_ref, out_ref, acc_ref, *, inv_n):
    b = pl.program_id(0)
    j = pl.program_id(1)
    nb = pl.num_programs(0)
    nj = pl.num_programs(1)
    step = b * nj + j

    @pl.when(step == 0)
    def _init():
        acc_ref[...] = jnp.zeros_like(acc_ref)

    x = x_ref[0]
    t = t_ref[0]

    m = jnp.max(x, axis=0)
    am = jnp.argmax(x, axis=0).astype(jnp.int32)

    s = jnp.sum(jnp.exp(x - m[None]), axis=0)
    lse = m + jnp.log(s)

    cls = lax.broadcasted_iota(jnp.int32, (_NCLS, _BY, 512), 0)
    h = (t[None] == cls)

    logit_t = jnp.sum(jnp.where(h, x, 0.0), axis=0)
    ce = lse - logit_t
    idex = (am != t).astype(jnp.float32)

    hf = h.astype(jnp.float32)
    gt_p = jnp.sum(hf, axis=1)
    fn_p = jnp.sum(hf * idex[None], axis=1)
    ce_p = jnp.sum(hf * ce[None], axis=1)

    acc_ref[0] += gt_p
    acc_ref[1] += fn_p
    acc_ref[2] += ce_p

    @pl.when(step == nb * nj - 1)
    def _fin():
        gt = jnp.sum(acc_ref[0], axis=1, keepdims=True)
        fn = jnp.sum(acc_ref[1], axis=1, keepdims=True)
        cs = jnp.sum(acc_ref[2], axis=1, keepdims=True)
        w = jnp.where(fn > 0, fn, 1.0) / jnp.where(gt > 0, gt, 1.0)
        loss = jnp.sum(w * cs) * inv_n
        out_ref[...] = jnp.full(out_ref.shape, loss, jnp.float32)


def kernel(input, target):
    B, C, H, W = input.shape
    nb = H // _BY
    n = B * H * W
    body = functools.partial(_rce_kernel, inv_n=1.0 / n)
    out = pl.pallas_call(
        body,
        grid=(B, nb),
        in_specs=[
            pl.BlockSpec((1, C, _BY, W), lambda b, j: (b, 0, j, 0)),
            pl.BlockSpec((1, _BY, W), lambda b, j: (b, j, 0)),
        ],
        out_specs=pl.BlockSpec((8, 128), lambda b, j: (0, 0)),
        out_shape=jax.ShapeDtypeStruct((8, 128), jnp.float32),
        scratch_shapes=[pltpu.VMEM((3, _NCLS, W), jnp.float32)],
    )(input, target)
    return out[0, 0]


# speedup vs baseline: 226.7697x; 1.2630x over previous
import functools

import jax
import jax.numpy as jnp
from jax.experimental import pallas as pl
from jax.experimental.pallas import tpu as pltpu

_NCLS = 19
_BY = 64


def _rce_kernel(x_ref, t_ref, out_ref, acc_ref, *, inv_n):
    b = pl.program_id(0)
    j = pl.program_id(1)
    nb = pl.num_programs(0)
    nj = pl.num_programs(1)
    step = b * nj + j

    @pl.when(step == 0)
    def _init():
        acc_ref[...] = jnp.zeros_like(acc_ref)

    x = x_ref[0]
    t = t_ref[0]

    m = x[0]
    for c in range(1, _NCLS):
        m = jnp.maximum(m, x[c])

    s = jnp.exp(x[0] - m)
    for c in range(1, _NCLS):
        s = s + jnp.exp(x[c] - m)
    lse = m + jnp.log(s)

    zero = jnp.zeros_like(m)
    one = jnp.ones_like(m)
    r = _BY // 8
    for c in range(_NCLS):
        xc = x[c]
        h = t == c
        wrong = h & (xc < m)
        gt_p = jnp.where(h, one, zero).reshape(r, 8, 512).sum(axis=0)
        fn_p = jnp.where(wrong, one, zero).reshape(r, 8, 512).sum(axis=0)
        ce_p = jnp.where(h, lse - xc, zero).reshape(r, 8, 512).sum(axis=0)
        acc_ref[0, c] += gt_p
        acc_ref[1, c] += fn_p
        acc_ref[2, c] += ce_p

    @pl.when(step == nb * nj - 1)
    def _fin():
        gt = jnp.sum(acc_ref[0], axis=(1, 2), keepdims=True)[:, 0]
        fn = jnp.sum(acc_ref[1], axis=(1, 2), keepdims=True)[:, 0]
        cs = jnp.sum(acc_ref[2], axis=(1, 2), keepdims=True)[:, 0]
        w = jnp.where(fn > 0, fn, 1.0) / jnp.where(gt > 0, gt, 1.0)
        loss = jnp.sum(w * cs) * inv_n
        out_ref[...] = jnp.full(out_ref.shape, loss, jnp.float32)


def kernel(input, target):
    B, C, H, W = input.shape
    nb = H // _BY
    n = B * H * W
    body = functools.partial(_rce_kernel, inv_n=1.0 / n)
    out = pl.pallas_call(
        body,
        grid=(B, nb),
        in_specs=[
            pl.BlockSpec((1, C, _BY, W), lambda b, j: (b, 0, j, 0)),
            pl.BlockSpec((1, _BY, W), lambda b, j: (b, j, 0)),
        ],
        out_specs=pl.BlockSpec((8, 128), lambda b, j: (0, 0)),
        out_shape=jax.ShapeDtypeStruct((8, 128), jnp.float32),
        scratch_shapes=[pltpu.VMEM((3, _NCLS, 8, W), jnp.float32)],
    )(input, target)
    return out[0, 0]
